# baseline (speedup 1.0000x reference)
import jax
import jax.numpy as jnp
from jax.experimental import pallas as pl
from jax.experimental.pallas import tpu as pltpu

_EPS = 1e-05
_NS_ITERS = 6
_SQUARINGS = 18
_G = 2
_HP = jax.lax.Precision.HIGHEST


def _zca_program(x_ref, o_ref, xc_ref, xhi_ref, xlo_ref):
    G, C, M = x_ref.shape
    eye = jnp.eye(C, dtype=jnp.float32)
    half3_eye = 1.5 * eye

    covs, gs, invgs = [], [], []
    for gi in range(G):
        xr = x_ref[gi]
        mu = jnp.mean(xr, axis=1, keepdims=True)
        xc_ref[gi] = xr - mu
        xc = xc_ref[gi]
        xhi = xc.astype(jnp.bfloat16)
        xhi_ref[gi] = xhi
        xlo_ref[gi] = (xc - xhi.astype(jnp.float32)).astype(jnp.bfloat16)
        cov = jax.lax.dot_general(
            xc, xc, (((1,), (1,)), ((), ())),
            preferred_element_type=jnp.float32,
        ) * (1.0 / M)
        cov = cov + _EPS * eye
        row_sums = jnp.sum(jnp.abs(cov), axis=0, keepdims=True)
        g = jnp.max(row_sums, axis=1, keepdims=True)
        covs.append(cov)
        gs.append(g)
        invgs.append(1.0 / g)

    ys = [covs[gi] * invgs[gi] for gi in range(G)]
    ts = [half3_eye - 0.5 * ys[gi] for gi in range(G)]
    zs = list(ts)
    ys = [jnp.dot(ys[gi], ts[gi], preferred_element_type=jnp.float32,
                  precision=_HP) for gi in range(G)]
    for it in range(1, _NS_ITERS):
        ts = [half3_eye - 0.5 * jnp.dot(zs[gi], ys[gi],
                                        preferred_element_type=jnp.float32,
                                        precision=_HP)
              for gi in range(G)]
        if it < _NS_ITERS - 1:
            ys = [jnp.dot(ys[gi], ts[gi], preferred_element_type=jnp.float32,
                          precision=_HP) for gi in range(G)]
        zs = [jnp.dot(ts[gi], zs[gi], preferred_element_type=jnp.float32,
                      precision=_HP) for gi in range(G)]
    s_fulls = [zs[gi] * jax.lax.rsqrt(gs[gi]) for gi in range(G)]

    qs = [(gs[gi] * eye - covs[gi]) * invgs[gi] for gi in range(G)]
    for i in range(_SQUARINGS):
        if i % 3 == 0:
            for gi in range(G):
                qmax = jnp.max(jnp.abs(qs[gi]), axis=1, keepdims=True)
                qmax = jnp.max(qmax, axis=0, keepdims=True)
                qs[gi] = qs[gi] * (1.0 / qmax)
        qs = [jnp.dot(qs[gi], qs[gi], preferred_element_type=jnp.float32,
                      precision=_HP) for gi in range(G)]

    for gi in range(G):
        q = qs[gi]
        tr = jnp.sum(q * eye, axis=0, keepdims=True)
        tr = jnp.sum(tr, axis=1, keepdims=True)
        p_min = q * (1.0 / tr)
        lam = jnp.sum(covs[gi] * p_min, axis=0, keepdims=True)
        lam = jnp.sum(lam, axis=1, keepdims=True)
        s_hat = s_fulls[gi] - jax.lax.rsqrt(lam) * p_min
        s_hi = s_hat.astype(jnp.bfloat16)
        s_lo = (s_hat - s_hi.astype(jnp.float32)).astype(jnp.bfloat16)
        xhi = xhi_ref[gi]
        xlo = xlo_ref[gi]
        o_ref[gi] = (
            jnp.dot(s_hi, xhi, preferred_element_type=jnp.float32)
            + jnp.dot(s_hi, xlo, preferred_element_type=jnp.float32)
            + jnp.dot(s_lo, xhi, preferred_element_type=jnp.float32)
        )


def _whiten_call(x):
    B, C, M = x.shape
    return pl.pallas_call(
        _zca_program,
        out_shape=jax.ShapeDtypeStruct((B, C, M), x.dtype),
        grid=(B // _G,),
        in_specs=[pl.BlockSpec((_G, C, M), lambda b: (b, 0, 0))],
        out_specs=pl.BlockSpec((_G, C, M), lambda b: (b, 0, 0)),
        scratch_shapes=[pltpu.VMEM((_G, C, M), jnp.float32),
                        pltpu.VMEM((_G, C, M), jnp.bfloat16),
                        pltpu.VMEM((_G, C, M), jnp.bfloat16)],
        compiler_params=pltpu.CompilerParams(
            dimension_semantics=("arbitrary",),
            vmem_limit_bytes=56 * 1024 * 1024,
        ),
        name="zca_whiten_g2",
    )(x)


def kernel(x):
    devs = jax.devices()
    if len(devs) >= 2 and x.shape[0] % (2 * _G) == 0:
        mesh = jax.make_mesh((2,), ("d",), devices=devs[:2])
        pspec = jax.sharding.PartitionSpec("d")
        x = jax.reshard(x, jax.sharding.NamedSharding(mesh, pspec))
        fn = jax.shard_map(_whiten_call, mesh=mesh,
                           in_specs=pspec, out_specs=pspec,
                           check_vma=False)
        return fn(x)
    return _whiten_call(x)


# speedup vs baseline: 3.0346x; 3.0346x over previous
import jax
import jax.numpy as jnp
from jax.experimental import pallas as pl
from jax.experimental.pallas import tpu as pltpu

_EPS = 1e-05
_NS_ITERS = 6
_SQUARINGS = 18
_G = 2
_HP = jax.lax.Precision.HIGHEST


def _zca_program(x_ref, o_ref, xc_ref, xhi_ref, xlo_ref):
    G, C, M = x_ref.shape
    eye = jnp.eye(C, dtype=jnp.float32)
    half3_eye = 1.5 * eye

    covs, gs, invgs = [], [], []
    for gi in range(G):
        xr = x_ref[gi]
        mu = jnp.mean(xr, axis=1, keepdims=True)
        xc_ref[gi] = xr - mu
        xc = xc_ref[gi]
        xhi = xc.astype(jnp.bfloat16)
        xhi_ref[gi] = xhi
        xlo_ref[gi] = (xc - xhi.astype(jnp.float32)).astype(jnp.bfloat16)
        cov = jax.lax.dot_general(
            xc, xc, (((1,), (1,)), ((), ())),
            preferred_element_type=jnp.float32,
        ) * (1.0 / M)
        cov = cov + _EPS * eye
        row_sums = jnp.sum(jnp.abs(cov), axis=0, keepdims=True)
        g = jnp.max(row_sums, axis=1, keepdims=True)
        covs.append(cov)
        gs.append(g)
        invgs.append(1.0 / g)

    ys = [covs[gi] * invgs[gi] for gi in range(G)]
    ts = [half3_eye - 0.5 * ys[gi] for gi in range(G)]
    zs = list(ts)
    ys = [jnp.dot(ys[gi], ts[gi], preferred_element_type=jnp.float32,
                  precision=_HP) for gi in range(G)]
    for it in range(1, _NS_ITERS):
        ts = [half3_eye - 0.5 * jnp.dot(zs[gi], ys[gi],
                                        preferred_element_type=jnp.float32,
                                        precision=_HP)
              for gi in range(G)]
        if it < _NS_ITERS - 1:
            ys = [jnp.dot(ys[gi], ts[gi], preferred_element_type=jnp.float32,
                          precision=_HP) for gi in range(G)]
        zs = [jnp.dot(ts[gi], zs[gi], preferred_element_type=jnp.float32,
                      precision=_HP) for gi in range(G)]
    s_fulls = [zs[gi] * jax.lax.rsqrt(gs[gi]) for gi in range(G)]

    qs = [(gs[gi] * eye - covs[gi]) * invgs[gi] for gi in range(G)]
    for i in range(_SQUARINGS):
        if i % 3 == 0:
            for gi in range(G):
                qmax = jnp.max(jnp.abs(qs[gi]), axis=1, keepdims=True)
                qmax = jnp.max(qmax, axis=0, keepdims=True)
                qs[gi] = qs[gi] * (1.0 / qmax)
        qs = [jnp.dot(qs[gi], qs[gi], preferred_element_type=jnp.float32,
                      precision=_HP) for gi in range(G)]

    for gi in range(G):
        q = qs[gi]
        tr = jnp.sum(q * eye, axis=0, keepdims=True)
        tr = jnp.sum(tr, axis=1, keepdims=True)
        p_min = q * (1.0 / tr)
        lam = jnp.sum(covs[gi] * p_min, axis=0, keepdims=True)
        lam = jnp.sum(lam, axis=1, keepdims=True)
        s_hat = s_fulls[gi] - jax.lax.rsqrt(lam) * p_min
        s_hi = s_hat.astype(jnp.bfloat16)
        s_lo = (s_hat - s_hi.astype(jnp.float32)).astype(jnp.bfloat16)
        xhi = xhi_ref[gi]
        xlo = xlo_ref[gi]
        o_ref[gi] = (
            jnp.dot(s_hi, xhi, preferred_element_type=jnp.float32)
            + jnp.dot(s_hi, xlo, preferred_element_type=jnp.float32)
            + jnp.dot(s_lo, xhi, preferred_element_type=jnp.float32)
        )


def kernel(x):
    B, C, M = x.shape
    return pl.pallas_call(
        _zca_program,
        out_shape=jax.ShapeDtypeStruct((B, C, M), x.dtype),
        grid=(B // _G,),
        in_specs=[pl.BlockSpec((_G, C, M), lambda b: (b, 0, 0))],
        out_specs=pl.BlockSpec((_G, C, M), lambda b: (b, 0, 0)),
        scratch_shapes=[pltpu.VMEM((_G, C, M), jnp.float32),
                        pltpu.VMEM((_G, C, M), jnp.bfloat16),
                        pltpu.VMEM((_G, C, M), jnp.bfloat16)],
        compiler_params=pltpu.CompilerParams(
            dimension_semantics=("parallel",),
            vmem_limit_bytes=56 * 1024 * 1024,
        ),
        name="zca_whiten_g2",
    )(x)
